# Initial kernel scaffold; baseline (speedup 1.0000x reference)
import jax
import jax.numpy as jnp
from jax.experimental import pallas as pl


def kernel(x, edge_index, batch, u, l0_W1, l0_b1, l0_W2, l0_b2, l0_W3, l0_b3, l1_W1, l1_b1, l1_W2, l1_b2, l1_W3, l1_b3, f_W1, f_b1, f_W2, f_b2, f_W3, f_b3):
    raise NotImplementedError("write your pallas kernel here")


# speedup vs baseline: 2.6669x; 2.6669x over previous
import functools

import jax
import jax.numpy as jnp
from jax import lax
from jax.experimental import pallas as pl
from jax.experimental.pallas import tpu as pltpu
from jax.experimental.pallas import tpu_sc as plsc

_N = 10000
_E = 320000
_D = 128
_H = 300
_L = 100
_G = 3
_LP = 128
_NW = 32
_PERW = _E // _NW
_CG = 80
_NP = 10240
_ROWS_PER_TILE = _NP // 16
_ZR = 128


def _worker_id():
    return lax.axis_index("s") * 2 + lax.axis_index("c")


def _make_gather(dp):
    mesh = plsc.VectorSubcoreMesh(core_axis_name="c", subcore_axis_name="s")

    @functools.partial(
        pl.kernel,
        out_type=(jax.ShapeDtypeStruct((_E, dp), jnp.float32),
                  jax.ShapeDtypeStruct((_E, dp), jnp.float32)),
        mesh=mesh,
        scratch_types=[
            pltpu.VMEM((_CG,), jnp.int32),
            pltpu.VMEM((_CG,), jnp.int32),
            pltpu.VMEM((_CG, dp), jnp.float32),
            pltpu.VMEM((_CG, dp), jnp.float32),
            pltpu.SemaphoreType.DMA,
            pltpu.SemaphoreType.DMA,
        ],
    )
    def gk(table, dsti, srci, outd, outs, div, siv, drv, srv, s1, s2):
        base = _worker_id() * _PERW

        def body(i, carry):
            off = base + i * _CG
            pltpu.sync_copy(dsti.at[pl.ds(off, _CG)], div)
            pltpu.sync_copy(srci.at[pl.ds(off, _CG)], siv)
            c1 = pltpu.async_copy(table.at[div], drv, s1)
            c2 = pltpu.async_copy(table.at[siv], srv, s2)
            c1.wait()
            c2.wait()
            pltpu.sync_copy(drv, outd.at[pl.ds(off, _CG)])
            pltpu.sync_copy(srv, outs.at[pl.ds(off, _CG)])
            return carry

        lax.fori_loop(0, _PERW // _CG, body, 0)

    return gk


_gather_d = _make_gather(_D)
_gather_lp = _make_gather(_LP)


def _make_scatter():
    mesh = plsc.VectorSubcoreMesh(core_axis_name="c", subcore_axis_name="s")

    @functools.partial(
        pl.kernel,
        out_type=jax.ShapeDtypeStruct((2 * _NP, _LP), jnp.float32),
        mesh=mesh,
        scratch_types=[
            pltpu.VMEM((_CG,), jnp.int32),
            pltpu.VMEM((_CG, _LP), jnp.float32),
            pltpu.VMEM((_ZR, _LP), jnp.float32),
            pltpu.VMEM_SHARED((_NP, _LP), jnp.float32),
        ],
    )
    def sk(m_hbm, dsti, parts, idxv, mv, zv, acc):
        c = lax.axis_index("c")
        s = lax.axis_index("s")
        base = (s * 2 + c) * _PERW
        zero16 = jnp.zeros((16,), jnp.float32)

        def zbody(i, carry):
            r = i // (_LP // 16)
            j = i % (_LP // 16)
            zv[r, pl.ds(j * 16, 16)] = zero16
            return carry

        lax.fori_loop(0, _ZR * (_LP // 16), zbody, 0)
        for k in range(_ROWS_PER_TILE // _ZR):
            pltpu.sync_copy(zv, acc.at[pl.ds(s * _ROWS_PER_TILE + k * _ZR, _ZR)])
        plsc.subcore_barrier()

        def body(i, carry):
            off = base + i * _CG
            pltpu.sync_copy(dsti.at[pl.ds(off, _CG)], idxv)
            pltpu.sync_copy(m_hbm.at[pl.ds(off, _CG)], mv)
            pltpu.sync_copy(mv, acc.at[idxv], add=True)
            return carry

        lax.fori_loop(0, _PERW // _CG, body, 0)
        plsc.subcore_barrier()
        for k in range(_ROWS_PER_TILE // _ZR):
            row0 = s * _ROWS_PER_TILE + k * _ZR
            pltpu.sync_copy(acc.at[pl.ds(row0, _ZR)], zv)
            pltpu.sync_copy(zv, parts.at[pl.ds(c * _NP + row0, _ZR)])

    return sk


_scatter = _make_scatter()


def _edge_mlp(gd, gs, w1d, w1s, b1, w2, b2, w3, b3):
    be = 2000
    din = gd.shape[1]

    def body(gd_ref, gs_ref, w1d_r, w1s_r, b1_r, w2_r, b2_r, w3_r, b3_r, o_ref):
        h = jnp.dot(gd_ref[...], w1d_r[...], preferred_element_type=jnp.float32)
        h += jnp.dot(gs_ref[...], w1s_r[...], preferred_element_type=jnp.float32)
        h = jnp.maximum(h + b1_r[...], 0.0)
        h = jnp.dot(h, w2_r[...], preferred_element_type=jnp.float32) + b2_r[...]
        h = jnp.maximum(h, 0.0)
        o_ref[...] = jnp.dot(h, w3_r[...], preferred_element_type=jnp.float32) + b3_r[...]

    full = lambda a: pl.BlockSpec(a.shape, lambda i: (0,) * a.ndim)
    return pl.pallas_call(
        body,
        grid=(_E // be,),
        in_specs=[
            pl.BlockSpec((be, din), lambda i: (i, 0)),
            pl.BlockSpec((be, din), lambda i: (i, 0)),
            full(w1d), full(w1s), full(b1), full(w2), full(b2), full(w3), full(b3),
        ],
        out_specs=pl.BlockSpec((be, _LP), lambda i: (i, 0)),
        out_shape=jax.ShapeDtypeStruct((_E, _LP), jnp.float32),
        compiler_params=pltpu.CompilerParams(
            dimension_semantics=("arbitrary",)),
    )(gd, gs, w1d, w1s, b1, w2, b2, w3, b3)


def _node_mean(parts):
    bn = 2048

    def body(p_ref, o_ref):
        sacc = p_ref[0] + p_ref[1]
        cnt = jnp.maximum(sacc[:, 100:101], 1.0)
        h = jnp.maximum(sacc / cnt, 0.0)
        col = lax.broadcasted_iota(jnp.int32, (bn, _LP), 1)
        o_ref[...] = jnp.where(col < _L, h, 0.0)

    return pl.pallas_call(
        body,
        grid=(_NP // bn,),
        in_specs=[pl.BlockSpec((2, bn, _LP), lambda i: (0, i, 0))],
        out_specs=pl.BlockSpec((bn, _LP), lambda i: (i, 0)),
        out_shape=jax.ShapeDtypeStruct((_NP, _LP), jnp.float32),
        compiler_params=pltpu.CompilerParams(
            dimension_semantics=("arbitrary",)),
    )(parts.reshape(2, _NP, _LP))


def _head(parts, u, fw1h, fw1u, fb1, fw2, fb2, fw3, fb3):
    def body(p_ref, u_ref, w1h_r, w1u_r, b1_r, w2_r, b2_r, w3_r, b3_r, o_ref):
        sacc = p_ref[0] + p_ref[1]
        cnt = jnp.maximum(sacc[:, 100:101], 1.0)
        h = jnp.maximum(sacc / cnt, 0.0)
        pooled = jnp.sum(h, axis=0, keepdims=True) * (1.0 / _N)
        z = jnp.dot(pooled, w1h_r[...], preferred_element_type=jnp.float32)
        z += jnp.dot(u_ref[...], w1u_r[...], preferred_element_type=jnp.float32)
        z = jnp.maximum(z + b1_r[...], 0.0)
        z = jnp.maximum(
            jnp.dot(z, w2_r[...], preferred_element_type=jnp.float32) + b2_r[...], 0.0)
        o_ref[...] = jnp.dot(z, w3_r[...], preferred_element_type=jnp.float32) + b3_r[...]

    return pl.pallas_call(
        body,
        out_shape=jax.ShapeDtypeStruct((1, 1), jnp.float32),
    )(parts.reshape(2, _NP, _LP), u, fw1h, fw1u, fb1, fw2, fb2, fw3, fb3)


def kernel(x, edge_index, batch, u,
           l0_W1, l0_b1, l0_W2, l0_b2, l0_W3, l0_b3,
           l1_W1, l1_b1, l1_W2, l1_b2, l1_W3, l1_b3,
           f_W1, f_b1, f_W2, f_b2, f_W3, f_b3):
    dst = edge_index[1]
    src = edge_index[0]

    w1d0 = l0_W1[:_D] - l0_W1[_D:]
    w1s0 = l0_W1[_D:]
    w1d1 = jnp.pad(l1_W1[:_L] - l1_W1[_L:], ((0, _LP - _L), (0, 0)))
    w1s1 = jnp.pad(l1_W1[_L:], ((0, _LP - _L), (0, 0)))
    onehot = jnp.zeros((_LP,), jnp.float32).at[_L].set(1.0)

    def pad3(w3, b3):
        w3p = jnp.pad(w3, ((0, 0), (0, _LP - _L)))
        b3p = jnp.pad(b3, (0, _LP - _L)) + onehot
        return w3p, b3p.reshape(1, _LP)

    w3p0, b3p0 = pad3(l0_W3, l0_b3)
    w3p1, b3p1 = pad3(l1_W3, l1_b3)
    fw1h = jnp.pad(f_W1[:_L], ((0, _LP - _L), (0, 0)))
    fw1u = f_W1[_L:]

    gd0, gs0 = _gather_d(x, dst, src)
    m0 = _edge_mlp(gd0, gs0, w1d0, w1s0, l0_b1.reshape(1, _H),
                   l0_W2, l0_b2.reshape(1, _H), w3p0, b3p0)
    parts0 = _scatter(m0, dst)
    h0 = _node_mean(parts0)
    gd1, gs1 = _gather_lp(h0, dst, src)
    m1 = _edge_mlp(gd1, gs1, w1d1, w1s1, l1_b1.reshape(1, _H),
                   l1_W2, l1_b2.reshape(1, _H), w3p1, b3p1)
    parts1 = _scatter(m1, dst)
    return _head(parts1, u, fw1h, fw1u, f_b1.reshape(1, _L),
                 f_W2, f_b2.reshape(1, _L), f_W3, f_b3.reshape(1, 1))


# speedup vs baseline: 3.4357x; 1.2883x over previous
import functools

import jax
import jax.numpy as jnp
from jax import lax
from jax.experimental import pallas as pl
from jax.experimental.pallas import tpu as pltpu
from jax.experimental.pallas import tpu_sc as plsc

_N = 10000
_E = 320000
_D = 128
_H = 300
_L = 100
_G = 3
_LP = 128
_NW = 32
_PERW = _E // _NW
_CG = 80
_NP = 10240
_ROWS_PER_TILE = _NP // 16
_ZR = 64


def _worker_id():
    return lax.axis_index("s") * 2 + lax.axis_index("c")


_CB = 400
_NSUB = _CB // _CG


def _make_gather(dp):
    mesh = plsc.VectorSubcoreMesh(core_axis_name="c", subcore_axis_name="s")

    @functools.partial(
        pl.kernel,
        out_type=(jax.ShapeDtypeStruct((_E, dp), jnp.float32),
                  jax.ShapeDtypeStruct((_E, dp), jnp.float32)),
        mesh=mesh,
        scratch_types=[
            pltpu.VMEM((_PERW,), jnp.int32),
            pltpu.VMEM((_PERW,), jnp.int32),
            pltpu.VMEM((_CB, dp), jnp.float32),
            pltpu.VMEM((_CB, dp), jnp.float32),
            pltpu.SemaphoreType.DMA,
            pltpu.SemaphoreType.DMA,
            pltpu.SemaphoreType.DMA,
            pltpu.SemaphoreType.DMA,
        ],
    )
    def gk(table, dsti, srci, outd, outs, div, siv, drv, srv, sgd, sgs, swd, sws):
        base = _worker_id() * _PERW
        pltpu.sync_copy(dsti.at[pl.ds(base, _PERW)], div)
        pltpu.sync_copy(srci.at[pl.ds(base, _PERW)], siv)

        def body(i, carry):
            @pl.when(i > 0)
            def _():
                pltpu.make_async_copy(drv, outd.at[pl.ds(base, _CB)], swd).wait()
                pltpu.make_async_copy(srv, outs.at[pl.ds(base, _CB)], sws).wait()

            cps = []
            for k in range(_NSUB):
                ixo = i * _CB + k * _CG
                cps.append(pltpu.async_copy(
                    table.at[div.at[pl.ds(ixo, _CG)]],
                    drv.at[pl.ds(k * _CG, _CG)], sgd))
            for k in range(_NSUB):
                ixo = i * _CB + k * _CG
                cps.append(pltpu.async_copy(
                    table.at[siv.at[pl.ds(ixo, _CG)]],
                    srv.at[pl.ds(k * _CG, _CG)], sgs))
            for cp in cps[:_NSUB]:
                cp.wait()
            pltpu.async_copy(drv, outd.at[pl.ds(base + i * _CB, _CB)], swd)
            for cp in cps[_NSUB:]:
                cp.wait()
            pltpu.async_copy(srv, outs.at[pl.ds(base + i * _CB, _CB)], sws)
            return carry

        lax.fori_loop(0, _PERW // _CB, body, 0)
        pltpu.make_async_copy(drv, outd.at[pl.ds(base, _CB)], swd).wait()
        pltpu.make_async_copy(srv, outs.at[pl.ds(base, _CB)], sws).wait()

    return gk


_gather_d = _make_gather(_D)
_gather_lp = _make_gather(_LP)


_CS = 80


def _make_scatter():
    mesh = plsc.VectorSubcoreMesh(core_axis_name="c", subcore_axis_name="s")

    @functools.partial(
        pl.kernel,
        out_type=jax.ShapeDtypeStruct((2 * _NP, _LP), jnp.float32),
        mesh=mesh,
        scratch_types=[
            pltpu.VMEM((_CS,), jnp.int32),
            pltpu.VMEM((_CS,), jnp.int32),
            pltpu.VMEM((_CS, _LP), jnp.float32),
            pltpu.VMEM((_CS, _LP), jnp.float32),
            pltpu.VMEM((_ZR, _LP), jnp.float32),
            pltpu.VMEM_SHARED((_NP, _LP), jnp.float32),
            pltpu.SemaphoreType.DMA,
            pltpu.SemaphoreType.DMA,
        ],
    )
    def sk(m_hbm, dsti, parts, ib0, ib1, mb0, mb1, zv, acc, sd0, sd1):
        c_ax = lax.axis_index("c")
        s_ax = lax.axis_index("s")
        base = (s_ax * 2 + c_ax) * _PERW
        ibufs = (ib0, ib1)
        mbufs = (mb0, mb1)
        sds = (sd0, sd1)
        zero16 = jnp.zeros((16,), jnp.float32)

        def zbody(i, carry):
            r = i // (_LP // 16)
            j = i % (_LP // 16)
            zv[r, pl.ds(j * 16, 16)] = zero16
            return carry

        lax.fori_loop(0, _ZR * (_LP // 16), zbody, 0)
        for k in range(_ROWS_PER_TILE // _ZR):
            pltpu.sync_copy(zv, acc.at[pl.ds(s_ax * _ROWS_PER_TILE + k * _ZR, _ZR)])
        plsc.subcore_barrier()

        nchunk = _PERW // _CS

        def load(p, ci):
            off = base + ci * _CS
            pltpu.async_copy(m_hbm.at[pl.ds(off, _CS)], mbufs[p], sds[p])
            pltpu.async_copy(dsti.at[pl.ds(off, _CS)], ibufs[p], sds[p])

        def drain(p, ci):
            off = base + ci * _CS
            pltpu.make_async_copy(m_hbm.at[pl.ds(off, _CS)], mbufs[p], sds[p]).wait()
            pltpu.make_async_copy(dsti.at[pl.ds(off, _CS)], ibufs[p], sds[p]).wait()

        def scadd(p):
            pltpu.sync_copy(mbufs[p], acc.at[ibufs[p]], add=True)

        npairs = nchunk // 2
        load(0, 0)
        load(1, 1)

        def body(it, carry):
            c = 2 * it
            drain(0, c)
            scadd(0)

            @pl.when(c + 2 < nchunk)
            def _():
                load(0, c + 2)

            drain(1, c + 1)
            scadd(1)

            @pl.when(c + 3 < nchunk)
            def _():
                load(1, c + 3)

            return carry

        lax.fori_loop(0, npairs, body, 0)
        drain(0, nchunk - 1)
        scadd(0)
        plsc.subcore_barrier()
        for k in range(_ROWS_PER_TILE // _ZR):
            row0 = s_ax * _ROWS_PER_TILE + k * _ZR
            pltpu.sync_copy(acc.at[pl.ds(row0, _ZR)], zv)
            pltpu.sync_copy(zv, parts.at[pl.ds(c_ax * _NP + row0, _ZR)])

    return sk


_scatter = _make_scatter()


def _edge_mlp(gd, gs, w1d, w1s, b1, w2, b2, w3, b3):
    be = 2000
    din = gd.shape[1]

    def body(gd_ref, gs_ref, w1d_r, w1s_r, b1_r, w2_r, b2_r, w3_r, b3_r, o_ref):
        bf = jnp.bfloat16
        h = jnp.dot(gd_ref[...].astype(bf), w1d_r[...],
                    preferred_element_type=jnp.float32)
        h += jnp.dot(gs_ref[...].astype(bf), w1s_r[...],
                     preferred_element_type=jnp.float32)
        h = jnp.maximum(h + b1_r[...], 0.0)
        h = jnp.dot(h.astype(bf), w2_r[...],
                    preferred_element_type=jnp.float32) + b2_r[...]
        h = jnp.maximum(h, 0.0)
        o_ref[...] = jnp.dot(h.astype(bf), w3_r[...],
                             preferred_element_type=jnp.float32) + b3_r[...]

    full = lambda a: pl.BlockSpec(a.shape, lambda i: (0,) * a.ndim)
    return pl.pallas_call(
        body,
        grid=(_E // be,),
        in_specs=[
            pl.BlockSpec((be, din), lambda i: (i, 0)),
            pl.BlockSpec((be, din), lambda i: (i, 0)),
            full(w1d), full(w1s), full(b1), full(w2), full(b2), full(w3), full(b3),
        ],
        out_specs=pl.BlockSpec((be, _LP), lambda i: (i, 0)),
        out_shape=jax.ShapeDtypeStruct((_E, _LP), jnp.float32),
        compiler_params=pltpu.CompilerParams(
            dimension_semantics=("arbitrary",)),
    )(gd, gs, w1d, w1s, b1, w2, b2, w3, b3)


def _node_mean(parts):
    bn = 2048

    def body(p_ref, o_ref):
        sacc = p_ref[0] + p_ref[1]
        cnt = jnp.maximum(sacc[:, 100:101], 1.0)
        h = jnp.maximum(sacc / cnt, 0.0)
        col = lax.broadcasted_iota(jnp.int32, (bn, _LP), 1)
        o_ref[...] = jnp.where(col < _L, h, 0.0)

    return pl.pallas_call(
        body,
        grid=(_NP // bn,),
        in_specs=[pl.BlockSpec((2, bn, _LP), lambda i: (0, i, 0))],
        out_specs=pl.BlockSpec((bn, _LP), lambda i: (i, 0)),
        out_shape=jax.ShapeDtypeStruct((_NP, _LP), jnp.float32),
        compiler_params=pltpu.CompilerParams(
            dimension_semantics=("arbitrary",)),
    )(parts.reshape(2, _NP, _LP))


def _head(parts, u, fw1h, fw1u, fb1, fw2, fb2, fw3, fb3):
    def body(p_ref, u_ref, w1h_r, w1u_r, b1_r, w2_r, b2_r, w3_r, b3_r, o_ref):
        sacc = p_ref[0] + p_ref[1]
        cnt = jnp.maximum(sacc[:, 100:101], 1.0)
        h = jnp.maximum(sacc / cnt, 0.0)
        pooled = jnp.sum(h, axis=0, keepdims=True) * (1.0 / _N)
        z = jnp.dot(pooled, w1h_r[...], preferred_element_type=jnp.float32)
        z += jnp.dot(u_ref[...], w1u_r[...], preferred_element_type=jnp.float32)
        z = jnp.maximum(z + b1_r[...], 0.0)
        z = jnp.maximum(
            jnp.dot(z, w2_r[...], preferred_element_type=jnp.float32) + b2_r[...], 0.0)
        o_ref[...] = jnp.dot(z, w3_r[...], preferred_element_type=jnp.float32) + b3_r[...]

    return pl.pallas_call(
        body,
        out_shape=jax.ShapeDtypeStruct((1, 1), jnp.float32),
    )(parts.reshape(2, _NP, _LP), u, fw1h, fw1u, fb1, fw2, fb2, fw3, fb3)


def kernel(x, edge_index, batch, u,
           l0_W1, l0_b1, l0_W2, l0_b2, l0_W3, l0_b3,
           l1_W1, l1_b1, l1_W2, l1_b2, l1_W3, l1_b3,
           f_W1, f_b1, f_W2, f_b2, f_W3, f_b3):
    dst = edge_index[1]
    src = edge_index[0]

    w1d0 = l0_W1[:_D] - l0_W1[_D:]
    w1s0 = l0_W1[_D:]
    w1d1 = jnp.pad(l1_W1[:_L] - l1_W1[_L:], ((0, _LP - _L), (0, 0)))
    w1s1 = jnp.pad(l1_W1[_L:], ((0, _LP - _L), (0, 0)))
    onehot = jnp.zeros((_LP,), jnp.float32).at[_L].set(1.0)

    def pad3(w3, b3):
        w3p = jnp.pad(w3, ((0, 0), (0, _LP - _L)))
        b3p = jnp.pad(b3, (0, _LP - _L)) + onehot
        return w3p, b3p.reshape(1, _LP)

    w3p0, b3p0 = pad3(l0_W3, l0_b3)
    w3p1, b3p1 = pad3(l1_W3, l1_b3)
    fw1h = jnp.pad(f_W1[:_L], ((0, _LP - _L), (0, 0)))
    fw1u = f_W1[_L:]

    bf = jnp.bfloat16
    gd0, gs0 = _gather_d(x, dst, src)
    m0 = _edge_mlp(gd0, gs0, w1d0.astype(bf), w1s0.astype(bf),
                   l0_b1.reshape(1, _H), l0_W2.astype(bf),
                   l0_b2.reshape(1, _H), w3p0.astype(bf), b3p0)
    parts0 = _scatter(m0, dst)
    h0 = _node_mean(parts0)
    gd1, gs1 = _gather_lp(h0, dst, src)
    m1 = _edge_mlp(gd1, gs1, w1d1.astype(bf), w1s1.astype(bf),
                   l1_b1.reshape(1, _H), l1_W2.astype(bf),
                   l1_b2.reshape(1, _H), w3p1.astype(bf), b3p1)
    parts1 = _scatter(m1, dst)
    return _head(parts1, u, fw1h, fw1u, f_b1.reshape(1, _L),
                 f_W2, f_b2.reshape(1, _L), f_W3, f_b3.reshape(1, 1))


# speedup vs baseline: 4.2697x; 1.2427x over previous
import functools

import jax
import jax.numpy as jnp
from jax import lax
from jax.experimental import pallas as pl
from jax.experimental.pallas import tpu as pltpu
from jax.experimental.pallas import tpu_sc as plsc

_N = 10000
_E = 320000
_EH = _E // 2
_D = 128
_H = 300
_L = 100
_G = 3
_LP = 128
_NW = 32
_PERW = _EH // _NW
_NP = 10240
_ROWS_PER_TILE = _NP // 16
_ZR = 64

_CB = 200
_GSUB = (80, 80, 40)
_CS = 40


def _worker_id():
    return lax.axis_index("s") * 2 + lax.axis_index("c")


def _make_gather(dp):
    mesh = plsc.VectorSubcoreMesh(core_axis_name="c", subcore_axis_name="s")

    @functools.partial(
        pl.kernel,
        out_type=(jax.ShapeDtypeStruct((_EH, dp), jnp.float32),
                  jax.ShapeDtypeStruct((_EH, dp), jnp.float32)),
        mesh=mesh,
        scratch_types=[
            pltpu.VMEM((_PERW,), jnp.int32),
            pltpu.VMEM((_PERW,), jnp.int32),
            pltpu.VMEM((_CB, dp), jnp.float32),
            pltpu.VMEM((_CB, dp), jnp.float32),
            pltpu.SemaphoreType.DMA,
            pltpu.SemaphoreType.DMA,
            pltpu.SemaphoreType.DMA,
            pltpu.SemaphoreType.DMA,
        ],
    )
    def gk(table, dsti, srci, outd, outs, div, siv, drv, srv, sgd, sgs, swd, sws):
        base = _worker_id() * _PERW
        pltpu.sync_copy(dsti.at[pl.ds(base, _PERW)], div)
        pltpu.sync_copy(srci.at[pl.ds(base, _PERW)], siv)

        def body(i, carry):
            @pl.when(i > 0)
            def _():
                pltpu.make_async_copy(drv, outd.at[pl.ds(base, _CB)], swd).wait()
                pltpu.make_async_copy(srv, outs.at[pl.ds(base, _CB)], sws).wait()

            cps = []
            for buf, ivec, sem in ((drv, div, sgd), (srv, siv, sgs)):
                o = 0
                for ln in _GSUB:
                    cps.append(pltpu.async_copy(
                        table.at[ivec.at[pl.ds(i * _CB + o, ln)]],
                        buf.at[pl.ds(o, ln)], sem))
                    o += ln
            nsub = len(_GSUB)
            for cp in cps[:nsub]:
                cp.wait()
            pltpu.async_copy(drv, outd.at[pl.ds(base + i * _CB, _CB)], swd)
            for cp in cps[nsub:]:
                cp.wait()
            pltpu.async_copy(srv, outs.at[pl.ds(base + i * _CB, _CB)], sws)
            return carry

        lax.fori_loop(0, _PERW // _CB, body, 0)
        pltpu.make_async_copy(drv, outd.at[pl.ds(base, _CB)], swd).wait()
        pltpu.make_async_copy(srv, outs.at[pl.ds(base, _CB)], sws).wait()

    return gk


_gather = _make_gather(_LP)


def _make_scatter():
    mesh = plsc.VectorSubcoreMesh(core_axis_name="c", subcore_axis_name="s")

    @functools.partial(
        pl.kernel,
        out_type=jax.ShapeDtypeStruct((2 * _NP, _LP), jnp.float32),
        mesh=mesh,
        scratch_types=[
            pltpu.VMEM((_CS,), jnp.int32),
            pltpu.VMEM((_CS,), jnp.int32),
            pltpu.VMEM((_CS, _LP), jnp.float32),
            pltpu.VMEM((_CS, _LP), jnp.float32),
            pltpu.VMEM((_ZR, _LP), jnp.float32),
            pltpu.VMEM_SHARED((_NP, _LP), jnp.float32),
            pltpu.SemaphoreType.DMA,
            pltpu.SemaphoreType.DMA,
        ],
    )
    def sk(m_hbm, dsti, parts, ib0, ib1, mb0, mb1, zv, acc, sd0, sd1):
        c_ax = lax.axis_index("c")
        s_ax = lax.axis_index("s")
        base = (s_ax * 2 + c_ax) * _PERW
        ibufs = (ib0, ib1)
        mbufs = (mb0, mb1)
        sds = (sd0, sd1)
        zero16 = jnp.zeros((16,), jnp.float32)

        def zbody(i, carry):
            r = i // (_LP // 16)
            j = i % (_LP // 16)
            zv[r, pl.ds(j * 16, 16)] = zero16
            return carry

        lax.fori_loop(0, _ZR * (_LP // 16), zbody, 0)
        for k in range(_ROWS_PER_TILE // _ZR):
            pltpu.sync_copy(zv, acc.at[pl.ds(s_ax * _ROWS_PER_TILE + k * _ZR, _ZR)])
        plsc.subcore_barrier()

        nchunk = _PERW // _CS

        def load(p, ci):
            off = base + ci * _CS
            pltpu.async_copy(m_hbm.at[pl.ds(off, _CS)], mbufs[p], sds[p])
            pltpu.async_copy(dsti.at[pl.ds(off, _CS)], ibufs[p], sds[p])

        def drain(p, ci):
            off = base + ci * _CS
            pltpu.make_async_copy(m_hbm.at[pl.ds(off, _CS)], mbufs[p], sds[p]).wait()
            pltpu.make_async_copy(dsti.at[pl.ds(off, _CS)], ibufs[p], sds[p]).wait()

        def scadd(p):
            pltpu.sync_copy(mbufs[p], acc.at[ibufs[p]], add=True)

        npairs = nchunk // 2
        load(0, 0)
        load(1, 1)

        def body(it, carry):
            c = 2 * it
            drain(0, c)
            scadd(0)

            @pl.when(c + 2 < nchunk)
            def _():
                load(0, c + 2)

            drain(1, c + 1)
            scadd(1)

            @pl.when(c + 3 < nchunk)
            def _():
                load(1, c + 3)

            return carry

        lax.fori_loop(0, npairs, body, 0)
        drain(0, nchunk - 1)
        scadd(0)
        plsc.subcore_barrier()
        for k in range(_ROWS_PER_TILE // _ZR):
            row0 = s_ax * _ROWS_PER_TILE + k * _ZR
            pltpu.sync_copy(acc.at[pl.ds(row0, _ZR)], zv)
            pltpu.sync_copy(zv, parts.at[pl.ds(c_ax * _NP + row0, _ZR)])

    return sk


_scatter = _make_scatter()


def _edge_mlp(gd, gs, w1d, w1s, b1, w2, b2, w3, b3):
    be = 2000
    din = gd.shape[1]

    def body(gd_ref, gs_ref, w1d_r, w1s_r, b1_r, w2_r, b2_r, w3_r, b3_r, o_ref):
        h = jnp.dot(gd_ref[...], w1d_r[...], preferred_element_type=jnp.float32)
        h += jnp.dot(gs_ref[...], w1s_r[...], preferred_element_type=jnp.float32)
        h = jnp.maximum(h + b1_r[...], 0.0)
        h = jnp.dot(h, w2_r[...], preferred_element_type=jnp.float32) + b2_r[...]
        h = jnp.maximum(h, 0.0)
        o_ref[...] = jnp.dot(h, w3_r[...], preferred_element_type=jnp.float32) + b3_r[...]

    full = lambda a: pl.BlockSpec(a.shape, lambda i: (0,) * a.ndim)
    return pl.pallas_call(
        body,
        grid=(_EH // be,),
        in_specs=[
            pl.BlockSpec((be, din), lambda i: (i, 0)),
            pl.BlockSpec((be, din), lambda i: (i, 0)),
            full(w1d), full(w1s), full(b1), full(w2), full(b2), full(w3), full(b3),
        ],
        out_specs=pl.BlockSpec((be, _LP), lambda i: (i, 0)),
        out_shape=jax.ShapeDtypeStruct((_EH, _LP), jnp.float32),
        compiler_params=pltpu.CompilerParams(
            dimension_semantics=("arbitrary",)),
    )(gd, gs, w1d, w1s, b1, w2, b2, w3, b3)


def _node_mean(parts_a, parts_b):
    bn = 2048

    def body(pa_ref, pb_ref, o_ref):
        sacc = pa_ref[0] + pa_ref[1] + pb_ref[0] + pb_ref[1]
        cnt = jnp.maximum(sacc[:, 100:101], 1.0)
        h = jnp.maximum(sacc / cnt, 0.0)
        col = lax.broadcasted_iota(jnp.int32, (bn, _LP), 1)
        o_ref[...] = jnp.where(col < _L, h, 0.0)

    spec = pl.BlockSpec((2, bn, _LP), lambda i: (0, i, 0))
    return pl.pallas_call(
        body,
        grid=(_NP // bn,),
        in_specs=[spec, spec],
        out_specs=pl.BlockSpec((bn, _LP), lambda i: (i, 0)),
        out_shape=jax.ShapeDtypeStruct((_NP, _LP), jnp.float32),
        compiler_params=pltpu.CompilerParams(
            dimension_semantics=("arbitrary",)),
    )(parts_a.reshape(2, _NP, _LP), parts_b.reshape(2, _NP, _LP))


def _head(parts_a, parts_b, u, fw1h, fw1u, fb1, fw2, fb2, fw3, fb3):
    def body(pa_ref, pb_ref, u_ref, w1h_r, w1u_r, b1_r, w2_r, b2_r, w3_r,
             b3_r, o_ref):
        sacc = pa_ref[0] + pa_ref[1] + pb_ref[0] + pb_ref[1]
        cnt = jnp.maximum(sacc[:, 100:101], 1.0)
        h = jnp.maximum(sacc / cnt, 0.0)
        pooled = jnp.sum(h, axis=0, keepdims=True) * (1.0 / _N)
        z = jnp.dot(pooled, w1h_r[...], preferred_element_type=jnp.float32)
        z += jnp.dot(u_ref[...], w1u_r[...], preferred_element_type=jnp.float32)
        z = jnp.maximum(z + b1_r[...], 0.0)
        z = jnp.maximum(
            jnp.dot(z, w2_r[...], preferred_element_type=jnp.float32) + b2_r[...], 0.0)
        o_ref[...] = jnp.dot(z, w3_r[...], preferred_element_type=jnp.float32) + b3_r[...]

    return pl.pallas_call(
        body,
        out_shape=jax.ShapeDtypeStruct((1, 1), jnp.float32),
    )(parts_a.reshape(2, _NP, _LP), parts_b.reshape(2, _NP, _LP),
      u, fw1h, fw1u, fb1, fw2, fb2, fw3, fb3)


def kernel(x, edge_index, batch, u,
           l0_W1, l0_b1, l0_W2, l0_b2, l0_W3, l0_b3,
           l1_W1, l1_b1, l1_W2, l1_b2, l1_W3, l1_b3,
           f_W1, f_b1, f_W2, f_b2, f_W3, f_b3):
    dst = edge_index[1]
    src = edge_index[0]
    dsts = (dst[:_EH], dst[_EH:])
    srcs = (src[:_EH], src[_EH:])

    w1d0 = l0_W1[:_D] - l0_W1[_D:]
    w1s0 = l0_W1[_D:]
    w1d1 = jnp.pad(l1_W1[:_L] - l1_W1[_L:], ((0, _LP - _L), (0, 0)))
    w1s1 = jnp.pad(l1_W1[_L:], ((0, _LP - _L), (0, 0)))
    onehot = jnp.zeros((_LP,), jnp.float32).at[_L].set(1.0)

    def pad3(w3, b3):
        w3p = jnp.pad(w3, ((0, 0), (0, _LP - _L)))
        b3p = jnp.pad(b3, (0, _LP - _L)) + onehot
        return w3p, b3p.reshape(1, _LP)

    w3p0, b3p0 = pad3(l0_W3, l0_b3)
    w3p1, b3p1 = pad3(l1_W3, l1_b3)
    fw1h = jnp.pad(f_W1[:_L], ((0, _LP - _L), (0, 0)))
    fw1u = f_W1[_L:]
    xp = jnp.concatenate([x, jnp.zeros((_N, _LP - _D), x.dtype)], axis=1) \
        if _D != _LP else x

    def layer(tab, w1d, w1s, b1, w2, b2, w3p, b3p):
        parts = []
        for half in (0, 1):
            gd, gs = _gather(tab, dsts[half], srcs[half])
            m = _edge_mlp(gd, gs, w1d, w1s, b1, w2, b2, w3p, b3p)
            parts.append(_scatter(m, dsts[half]))
        return parts

    pa0, pb0 = layer(xp, w1d0, w1s0, l0_b1.reshape(1, _H),
                     l0_W2, l0_b2.reshape(1, _H), w3p0, b3p0)
    h0 = _node_mean(pa0, pb0)
    pa1, pb1 = layer(h0, w1d1, w1s1, l1_b1.reshape(1, _H),
                     l1_W2, l1_b2.reshape(1, _H), w3p1, b3p1)
    return _head(pa1, pb1, u, fw1h, fw1u, f_b1.reshape(1, _L),
                 f_W2, f_b2.reshape(1, _L), f_W3, f_b3.reshape(1, 1))


# speedup vs baseline: 4.5402x; 1.0633x over previous
import functools

import jax
import jax.numpy as jnp
from jax import lax
from jax.experimental import pallas as pl
from jax.experimental.pallas import tpu as pltpu
from jax.experimental.pallas import tpu_sc as plsc

_N = 10000
_E = 320000
_EH0 = 166400
_EH1 = _E - _EH0
_D = 128
_H = 300
_L = 100
_G = 3
_LP = 128
_NW = 32
_NP = 10240
_ROWS_PER_TILE = _NP // 16
_ZR = 64

_CB = 400
_CG = 80
_CS = 80


def _worker_id():
    return lax.axis_index("s") * 2 + lax.axis_index("c")


def _make_gather(eh, dp):
    perw = eh // _NW
    mesh = plsc.VectorSubcoreMesh(core_axis_name="c", subcore_axis_name="s")

    @functools.partial(
        pl.kernel,
        out_type=(jax.ShapeDtypeStruct((eh, dp), jnp.float32),
                  jax.ShapeDtypeStruct((eh, dp), jnp.float32)),
        mesh=mesh,
        scratch_types=[
            pltpu.VMEM((perw,), jnp.int32),
            pltpu.VMEM((perw,), jnp.int32),
            pltpu.VMEM((_CB, dp), jnp.float32),
            pltpu.VMEM((_CB, dp), jnp.float32),
            pltpu.SemaphoreType.DMA,
            pltpu.SemaphoreType.DMA,
            pltpu.SemaphoreType.DMA,
            pltpu.SemaphoreType.DMA,
        ],
    )
    def gk(table, dsti, srci, outd, outs, div, siv, drv, srv, sgd, sgs, swd, sws):
        base = _worker_id() * perw
        pltpu.sync_copy(dsti.at[pl.ds(base, perw)], div)
        pltpu.sync_copy(srci.at[pl.ds(base, perw)], siv)
        nsub = _CB // _CG

        def body(i, carry):
            @pl.when(i > 0)
            def _():
                pltpu.make_async_copy(drv, outd.at[pl.ds(base, _CB)], swd).wait()
                pltpu.make_async_copy(srv, outs.at[pl.ds(base, _CB)], sws).wait()

            cps = []
            for buf, ivec, sem in ((drv, div, sgd), (srv, siv, sgs)):
                for k in range(nsub):
                    o = k * _CG
                    cps.append(pltpu.async_copy(
                        table.at[ivec.at[pl.ds(i * _CB + o, _CG)]],
                        buf.at[pl.ds(o, _CG)], sem))
            for cp in cps[:nsub]:
                cp.wait()
            pltpu.async_copy(drv, outd.at[pl.ds(base + i * _CB, _CB)], swd)
            for cp in cps[nsub:]:
                cp.wait()
            pltpu.async_copy(srv, outs.at[pl.ds(base + i * _CB, _CB)], sws)
            return carry

        lax.fori_loop(0, perw // _CB, body, 0)
        pltpu.make_async_copy(drv, outd.at[pl.ds(base, _CB)], swd).wait()
        pltpu.make_async_copy(srv, outs.at[pl.ds(base, _CB)], sws).wait()

    return gk


_gathers = (_make_gather(_EH0, _LP), _make_gather(_EH1, _LP))


def _make_scatter(eh):
    perw = eh // _NW
    mesh = plsc.VectorSubcoreMesh(core_axis_name="c", subcore_axis_name="s")

    @functools.partial(
        pl.kernel,
        out_type=jax.ShapeDtypeStruct((2 * _NP, _LP), jnp.float32),
        mesh=mesh,
        scratch_types=[
            pltpu.VMEM((_CS,), jnp.int32),
            pltpu.VMEM((_CS,), jnp.int32),
            pltpu.VMEM((_CS, _LP), jnp.float32),
            pltpu.VMEM((_CS, _LP), jnp.float32),
            pltpu.VMEM((_ZR, _LP), jnp.float32),
            pltpu.VMEM_SHARED((_NP, _LP), jnp.float32),
            pltpu.SemaphoreType.DMA,
            pltpu.SemaphoreType.DMA,
        ],
    )
    def sk(m_hbm, dsti, parts, ib0, ib1, mb0, mb1, zv, acc, sd0, sd1):
        c_ax = lax.axis_index("c")
        s_ax = lax.axis_index("s")
        base = (s_ax * 2 + c_ax) * perw
        ibufs = (ib0, ib1)
        mbufs = (mb0, mb1)
        sds = (sd0, sd1)
        zero16 = jnp.zeros((16,), jnp.float32)

        def zbody(i, carry):
            r = i // (_LP // 16)
            j = i % (_LP // 16)
            zv[r, pl.ds(j * 16, 16)] = zero16
            return carry

        lax.fori_loop(0, _ZR * (_LP // 16), zbody, 0)
        for k in range(_ROWS_PER_TILE // _ZR):
            pltpu.sync_copy(zv, acc.at[pl.ds(s_ax * _ROWS_PER_TILE + k * _ZR, _ZR)])
        plsc.subcore_barrier()

        nchunk = perw // _CS

        def load(p, ci):
            off = base + ci * _CS
            pltpu.async_copy(m_hbm.at[pl.ds(off, _CS)], mbufs[p], sds[p])
            pltpu.async_copy(dsti.at[pl.ds(off, _CS)], ibufs[p], sds[p])

        def drain(p, ci):
            off = base + ci * _CS
            pltpu.make_async_copy(m_hbm.at[pl.ds(off, _CS)], mbufs[p], sds[p]).wait()
            pltpu.make_async_copy(dsti.at[pl.ds(off, _CS)], ibufs[p], sds[p]).wait()

        def scadd(p):
            pltpu.sync_copy(mbufs[p], acc.at[ibufs[p]], add=True)

        npairs = nchunk // 2
        load(0, 0)
        load(1, 1)

        def body(it, carry):
            c = 2 * it
            drain(0, c)
            scadd(0)

            @pl.when(c + 2 < nchunk)
            def _():
                load(0, c + 2)

            drain(1, c + 1)
            scadd(1)

            @pl.when(c + 3 < nchunk)
            def _():
                load(1, c + 3)

            return carry

        lax.fori_loop(0, npairs, body, 0)
        if nchunk % 2:
            drain(0, nchunk - 1)
            scadd(0)
        plsc.subcore_barrier()
        for k in range(_ROWS_PER_TILE // _ZR):
            row0 = s_ax * _ROWS_PER_TILE + k * _ZR
            pltpu.sync_copy(acc.at[pl.ds(row0, _ZR)], zv)
            pltpu.sync_copy(zv, parts.at[pl.ds(c_ax * _NP + row0, _ZR)])

    return sk


_scatters = (_make_scatter(_EH0), _make_scatter(_EH1))


def _edge_mlp(gd, gs, w1d, w1s, b1, w2, b2, w3, b3):
    be = 2560
    eh = gd.shape[0]
    din = gd.shape[1]

    def body(gd_ref, gs_ref, w1d_r, w1s_r, b1_r, w2_r, b2_r, w3_r, b3_r, o_ref):
        h = jnp.dot(gd_ref[...], w1d_r[...], preferred_element_type=jnp.float32)
        h += jnp.dot(gs_ref[...], w1s_r[...], preferred_element_type=jnp.float32)
        h = jnp.maximum(h + b1_r[...], 0.0)
        h = jnp.dot(h, w2_r[...], preferred_element_type=jnp.float32) + b2_r[...]
        h = jnp.maximum(h, 0.0)
        o_ref[...] = jnp.dot(h, w3_r[...], preferred_element_type=jnp.float32) + b3_r[...]

    full = lambda a: pl.BlockSpec(a.shape, lambda i: (0,) * a.ndim)
    return pl.pallas_call(
        body,
        grid=(eh // be,),
        in_specs=[
            pl.BlockSpec((be, din), lambda i: (i, 0)),
            pl.BlockSpec((be, din), lambda i: (i, 0)),
            full(w1d), full(w1s), full(b1), full(w2), full(b2), full(w3), full(b3),
        ],
        out_specs=pl.BlockSpec((be, _LP), lambda i: (i, 0)),
        out_shape=jax.ShapeDtypeStruct((eh, _LP), jnp.float32),
        compiler_params=pltpu.CompilerParams(
            dimension_semantics=("arbitrary",)),
    )(gd, gs, w1d, w1s, b1, w2, b2, w3, b3)


def _node_mean(parts_a, parts_b):
    bn = 2048

    def body(pa_ref, pb_ref, o_ref):
        sacc = pa_ref[0] + pa_ref[1] + pb_ref[0] + pb_ref[1]
        cnt = jnp.maximum(sacc[:, 100:101], 1.0)
        h = jnp.maximum(sacc / cnt, 0.0)
        col = lax.broadcasted_iota(jnp.int32, (bn, _LP), 1)
        o_ref[...] = jnp.where(col < _L, h, 0.0)

    spec = pl.BlockSpec((2, bn, _LP), lambda i: (0, i, 0))
    return pl.pallas_call(
        body,
        grid=(_NP // bn,),
        in_specs=[spec, spec],
        out_specs=pl.BlockSpec((bn, _LP), lambda i: (i, 0)),
        out_shape=jax.ShapeDtypeStruct((_NP, _LP), jnp.float32),
        compiler_params=pltpu.CompilerParams(
            dimension_semantics=("arbitrary",)),
    )(parts_a.reshape(2, _NP, _LP), parts_b.reshape(2, _NP, _LP))


def _head(parts_a, parts_b, u, fw1h, fw1u, fb1, fw2, fb2, fw3, fb3):
    def body(pa_ref, pb_ref, u_ref, w1h_r, w1u_r, b1_r, w2_r, b2_r, w3_r,
             b3_r, o_ref):
        sacc = pa_ref[0] + pa_ref[1] + pb_ref[0] + pb_ref[1]
        cnt = jnp.maximum(sacc[:, 100:101], 1.0)
        h = jnp.maximum(sacc / cnt, 0.0)
        pooled = jnp.sum(h, axis=0, keepdims=True) * (1.0 / _N)
        z = jnp.dot(pooled, w1h_r[...], preferred_element_type=jnp.float32)
        z += jnp.dot(u_ref[...], w1u_r[...], preferred_element_type=jnp.float32)
        z = jnp.maximum(z + b1_r[...], 0.0)
        z = jnp.maximum(
            jnp.dot(z, w2_r[...], preferred_element_type=jnp.float32) + b2_r[...], 0.0)
        o_ref[...] = jnp.dot(z, w3_r[...], preferred_element_type=jnp.float32) + b3_r[...]

    return pl.pallas_call(
        body,
        out_shape=jax.ShapeDtypeStruct((1, 1), jnp.float32),
    )(parts_a.reshape(2, _NP, _LP), parts_b.reshape(2, _NP, _LP),
      u, fw1h, fw1u, fb1, fw2, fb2, fw3, fb3)


def kernel(x, edge_index, batch, u,
           l0_W1, l0_b1, l0_W2, l0_b2, l0_W3, l0_b3,
           l1_W1, l1_b1, l1_W2, l1_b2, l1_W3, l1_b3,
           f_W1, f_b1, f_W2, f_b2, f_W3, f_b3):
    dst = edge_index[1]
    src = edge_index[0]
    dsts = (dst[:_EH0], dst[_EH0:])
    srcs = (src[:_EH0], src[_EH0:])

    w1d0 = l0_W1[:_D] - l0_W1[_D:]
    w1s0 = l0_W1[_D:]
    w1d1 = jnp.pad(l1_W1[:_L] - l1_W1[_L:], ((0, _LP - _L), (0, 0)))
    w1s1 = jnp.pad(l1_W1[_L:], ((0, _LP - _L), (0, 0)))
    onehot = jnp.zeros((_LP,), jnp.float32).at[_L].set(1.0)

    def pad3(w3, b3):
        w3p = jnp.pad(w3, ((0, 0), (0, _LP - _L)))
        b3p = jnp.pad(b3, (0, _LP - _L)) + onehot
        return w3p, b3p.reshape(1, _LP)

    w3p0, b3p0 = pad3(l0_W3, l0_b3)
    w3p1, b3p1 = pad3(l1_W3, l1_b3)
    fw1h = jnp.pad(f_W1[:_L], ((0, _LP - _L), (0, 0)))
    fw1u = f_W1[_L:]
    xp = jnp.concatenate([x, jnp.zeros((_N, _LP - _D), x.dtype)], axis=1) \
        if _D != _LP else x

    def layer(tab, w1d, w1s, b1, w2, b2, w3p, b3p):
        parts = []
        for half in (0, 1):
            gd, gs = _gathers[half](tab, dsts[half], srcs[half])
            m = _edge_mlp(gd, gs, w1d, w1s, b1, w2, b2, w3p, b3p)
            parts.append(_scatters[half](m, dsts[half]))
        return parts

    pa0, pb0 = layer(xp, w1d0, w1s0, l0_b1.reshape(1, _H),
                     l0_W2, l0_b2.reshape(1, _H), w3p0, b3p0)
    h0 = _node_mean(pa0, pb0)
    pa1, pb1 = layer(h0, w1d1, w1s1, l1_b1.reshape(1, _H),
                     l1_W2, l1_b2.reshape(1, _H), w3p1, b3p1)
    return _head(pa1, pb1, u, fw1h, fw1u, f_b1.reshape(1, _L),
                 f_W2, f_b2.reshape(1, _L), f_W3, f_b3.reshape(1, 1))
